# baseline (speedup 1.0000x reference)
import functools

import jax
import jax.numpy as jnp
from jax import lax
from jax.experimental import pallas as pl
from jax.experimental.pallas import tpu as pltpu
from jax.experimental.pallas import tpu_sc as plsc

N, K, D = 16384, 100000, 64
NC, NS, LANES = 2, 16, 16
NW = NC * NS
ROWS_PER_W = N // NW
CHUNK = 128
NCHUNK = ROWS_PER_W // CHUNK
COLV = D // LANES


def _sc_body(feat_hbm, idx_hbm, ex_hbm, var_hbm, out_hbm,
             idx_v, feat_v, ex_v, var_v, acc_v, sem):
    c = lax.axis_index("c")
    s = lax.axis_index("s")
    wid = s * NC + c

    pltpu.sync_copy(idx_hbm.at[wid], idx_v)

    acc = jnp.zeros((LANES,), jnp.float32)
    for j in range(NCHUNK):
        row0 = wid * ROWS_PER_W + j * CHUNK
        pltpu.sync_copy(feat_hbm.at[pl.ds(row0, CHUNK)], feat_v)
        pltpu.async_copy(ex_hbm.at[idx_v.at[j]], ex_v, sem).wait()
        pltpu.async_copy(var_hbm.at[idx_v.at[j]], var_v, sem).wait()

        def row_body(r, a):
            for q in range(COLV):
                f = feat_v[r, pl.ds(q * LANES, LANES)]
                e = ex_v[r, pl.ds(q * LANES, LANES)]
                v = var_v[r, pl.ds(q * LANES, LANES)]
                a = a + jnp.abs(f - e) / v
            return a

        acc = lax.fori_loop(0, CHUNK, row_body, acc)

    acc_v[...] = acc
    pltpu.sync_copy(acc_v, out_hbm.at[wid])


@jax.jit
def _sc_loss(feat, idx3, ex, var):
    mesh = plsc.VectorSubcoreMesh(core_axis_name="c", subcore_axis_name="s")
    partials = pl.kernel(
        _sc_body,
        mesh=mesh,
        out_type=jax.ShapeDtypeStruct((NW, LANES), jnp.float32),
        compiler_params=pltpu.CompilerParams(use_tc_tiling_on_sc=False),
        scratch_types=[
            pltpu.VMEM((NCHUNK, CHUNK), jnp.int32),
            pltpu.VMEM((CHUNK, D), jnp.float32),
            pltpu.VMEM((CHUNK, D), jnp.float32),
            pltpu.VMEM((CHUNK, D), jnp.float32),
            pltpu.VMEM((LANES,), jnp.float32),
            pltpu.SemaphoreType.DMA,
        ],
    )(feat, idx3, ex, var)
    return jnp.sum(partials) / jnp.float32(N)


def kernel(action_features_actionframes, action_idxs_actionframes,
           examplars, examplars_variances):
    idx3 = action_idxs_actionframes.astype(jnp.int32).reshape(NW, NCHUNK, CHUNK)
    return _sc_loss(action_features_actionframes, idx3,
                    examplars, examplars_variances)
